# Initial kernel scaffold; baseline (speedup 1.0000x reference)
import jax
import jax.numpy as jnp
from jax.experimental import pallas as pl


def kernel(x):
    raise NotImplementedError("write your pallas kernel here")


# speedup vs baseline: 28.8140x; 28.8140x over previous
import functools

import jax
import jax.numpy as jnp
from jax.experimental import pallas as pl
from jax.experimental.pallas import tpu as pltpu

_B, _C, _H, _W = 64, 3, 224, 224
_SHIFT = 28
_CUT = 112
_PAD = _H + 2 * _SHIFT


def _aug_params(dtype):
    key = jax.random.key(42)
    k1, k2, k3, k4, k5, k6, k7 = jax.random.split(key, 7)
    b = (jax.random.uniform(k1, (_B, 1, 1, 1), dtype=dtype) - 0.5).reshape(_B)
    sat = (jax.random.uniform(k2, (_B, 1, 1, 1), dtype=dtype) * 2.0).reshape(_B)
    con = (jax.random.uniform(k3, (_B, 1, 1, 1), dtype=dtype) + 0.5).reshape(_B)
    tx = jax.random.randint(k4, (_B, 1, 1), -_SHIFT, _SHIFT + 1).reshape(_B)
    ty = jax.random.randint(k5, (_B, 1, 1), -_SHIFT, _SHIFT + 1).reshape(_B)
    ox = jax.random.randint(k6, (_B, 1, 1), 0, _H + (1 - _CUT % 2)).reshape(_B)
    oy = jax.random.randint(k7, (_B, 1, 1), 0, _W + (1 - _CUT % 2)).reshape(_B)
    alpha = con * sat
    beta = con * (1.0 - sat)
    omc = 1.0 - con
    pf = jnp.stack([alpha, beta, b, omc])
    pi = jnp.stack([tx, ty, ox, oy]).astype(jnp.int32)
    return pf, pi


def _body(pf_ref, pi_ref, x_ref, o_ref):
    s = pl.program_id(0)
    x = x_ref[0]
    alpha = pf_ref[0, s]
    beta = pf_ref[1, s]
    bb = pf_ref[2, s]
    omc = pf_ref[3, s]
    tx = pi_ref[0, s]
    ty = pi_ref[1, s]
    ox = pi_ref[2, s]
    oy = pi_ref[3, s]

    m0 = jnp.mean(x)
    mu = (x[0] + x[1] + x[2]) * (1.0 / 3.0)
    gamma = bb + omc * m0
    x3 = alpha * x + beta * mu[None] + gamma

    v = pltpu.roll(pltpu.roll(x3, (_H - tx) % _H, axis=1),
                   (_W - ty) % _W, axis=2)

    row = jax.lax.broadcasted_iota(jnp.int32, (_H, _W), 0)
    col = jax.lax.broadcasted_iota(jnp.int32, (_H, _W), 1)
    half = _CUT // 2
    cut = ((row >= ox - half) & (row <= ox + half - 1)
           & (col >= oy - half) & (col <= oy + half - 1))
    valid = ((row + tx >= 0) & (row + tx < _H)
             & (col + ty >= 0) & (col + ty < _W) & ~cut)
    o_ref[0] = jnp.where(valid[None], v, 0.0)


@jax.jit
def kernel(x):
    pf, pi = _aug_params(x.dtype)
    return pl.pallas_call(
        _body,
        grid=(_B,),
        in_specs=[
            pl.BlockSpec(memory_space=pltpu.SMEM),
            pl.BlockSpec(memory_space=pltpu.SMEM),
            pl.BlockSpec((1, _C, _H, _W), lambda s: (s, 0, 0, 0)),
        ],
        out_specs=pl.BlockSpec((1, _C, _H, _W), lambda s: (s, 0, 0, 0)),
        out_shape=jax.ShapeDtypeStruct((_B, _C, _H, _W), x.dtype),
        compiler_params=pltpu.CompilerParams(
            dimension_semantics=("arbitrary",),
        ),
    )(pf, pi, x)
